# Initial kernel scaffold; baseline (speedup 1.0000x reference)
import jax
import jax.numpy as jnp
from jax.experimental import pallas as pl


def kernel(x, edge_index, edge_attr, u, W0, b0, W1, b1):
    raise NotImplementedError("write your pallas kernel here")


# speedup vs baseline: 8.0764x; 8.0764x over previous
import functools

import jax
import jax.numpy as jnp
from jax import lax
from jax.experimental import pallas as pl
from jax.experimental.pallas import tpu as pltpu
from jax.experimental.pallas import tpu_sc as plsc

N_NODES = 10000
N_EDGES = 320000
D_EDGE = 16
D_FEAT = 128

NC = 2
NS = 16
NW = NC * NS

CHUNK = 128
N_CHUNKS = N_EDGES // CHUNK
CPW = 80
PAD_CHUNKS = CPW * NW
CB = 10
NB = CPW // CB
EB = CB * CHUNK
STRIPE = 624
LAST_EXTRA = N_NODES - STRIPE * NS


def _sc_segment_sums(attr, ridx, cidx, zeros):
    mesh = plsc.VectorSubcoreMesh(core_axis_name="c", subcore_axis_name="s")

    @functools.partial(
        pl.kernel,
        mesh=mesh,
        out_type=jax.ShapeDtypeStruct((NC, 2, N_NODES, D_EDGE), jnp.float32),
        scratch_types=[
            pltpu.VMEM_SHARED((N_NODES, D_EDGE), jnp.float32),
            pltpu.VMEM_SHARED((N_NODES, D_EDGE), jnp.float32),
            pltpu.VMEM((2, EB, D_EDGE), jnp.float32),
            pltpu.VMEM((CPW, CHUNK), jnp.int32),
            pltpu.VMEM((CPW, CHUNK), jnp.int32),
            pltpu.SemaphoreType.DMA,
            pltpu.SemaphoreType.DMA,
        ],
        compiler_params=pltpu.CompilerParams(use_tc_tiling_on_sc=False),
    )
    def k(attr_hbm, ridx_hbm, cidx_hbm, z_hbm, out_hbm,
          rtab, ctab, abuf, rv, cv, sem0, sem1):
        c = lax.axis_index("c")
        s = lax.axis_index("s")
        w = c * NS + s

        soff = pl.multiple_of(s * STRIPE, 8)
        pltpu.sync_copy(z_hbm, rtab.at[pl.ds(soff, STRIPE)])
        pltpu.sync_copy(z_hbm, ctab.at[pl.ds(soff, STRIPE)])

        @pl.when(s == NS - 1)
        def _():
            pltpu.sync_copy(z_hbm.at[pl.ds(0, LAST_EXTRA)],
                            rtab.at[pl.ds(STRIPE * NS, LAST_EXTRA)])
            pltpu.sync_copy(z_hbm.at[pl.ds(0, LAST_EXTRA)],
                            ctab.at[pl.ds(STRIPE * NS, LAST_EXTRA)])

        plsc.subcore_barrier()

        row0 = pl.multiple_of(w * CPW, 8)
        pltpu.sync_copy(ridx_hbm.at[pl.ds(row0, CPW)], rv)
        pltpu.sync_copy(cidx_hbm.at[pl.ds(row0, CPW)], cv)

        def estart(b):
            e = jnp.minimum(w * (CPW * CHUNK) + b * EB, N_EDGES - EB)
            return pl.multiple_of(e, 8)

        sems = (sem0, sem1)
        pending = pltpu.async_copy(attr_hbm.at[pl.ds(estart(0), EB)],
                                   abuf.at[0], sems[0])
        for b in range(NB):
            if b + 1 < NB:
                nxt = pltpu.async_copy(attr_hbm.at[pl.ds(estart(b + 1), EB)],
                                       abuf.at[(b + 1) % 2], sems[(b + 1) % 2])
            pending.wait()
            par = b % 2

            @pl.when(w * CPW + (b + 1) * CB <= N_CHUNKS)
            def _(par=par, b=b):
                def chunk_body(j, _):
                    src = abuf.at[par, pl.ds(j * CHUNK, CHUNK)]
                    pltpu.sync_copy(src, rtab.at[rv.at[b * CB + j]], add=True)
                    pltpu.sync_copy(src, ctab.at[cv.at[b * CB + j]], add=True)
                    return 0

                lax.fori_loop(0, CB, chunk_body, 0)

            if b + 1 < NB:
                pending = nxt

        plsc.subcore_barrier()

        pltpu.sync_copy(rtab.at[pl.ds(soff, STRIPE)],
                        out_hbm.at[c, 0, pl.ds(soff, STRIPE)])
        pltpu.sync_copy(ctab.at[pl.ds(soff, STRIPE)],
                        out_hbm.at[c, 1, pl.ds(soff, STRIPE)])

        @pl.when(s == NS - 1)
        def _():
            pltpu.sync_copy(rtab.at[pl.ds(STRIPE * NS, LAST_EXTRA)],
                            out_hbm.at[c, 0, pl.ds(STRIPE * NS, LAST_EXTRA)])
            pltpu.sync_copy(ctab.at[pl.ds(STRIPE * NS, LAST_EXTRA)],
                            out_hbm.at[c, 1, pl.ds(STRIPE * NS, LAST_EXTRA)])

    return k(attr, ridx, cidx, zeros)


def _mlp_body(parts_ref, x_ref, w0c_ref, w0r_ref, w0x_ref, beff_ref,
              w1t_ref, b1_ref, out_ref):
    row_agg = parts_ref[0, 0] + parts_ref[1, 0]
    col_agg = parts_ref[0, 1] + parts_ref[1, 1]
    h = jnp.dot(x_ref[...], w0x_ref[...], preferred_element_type=jnp.float32)
    h = h + jnp.dot(col_agg, w0c_ref[...], preferred_element_type=jnp.float32)
    h = h + jnp.dot(row_agg, w0r_ref[...], preferred_element_type=jnp.float32)
    h = h + beff_ref[...]
    h = jnp.where(h >= 0, h, 0.2 * h)
    out_ref[...] = (
        jnp.dot(h, w1t_ref[...], preferred_element_type=jnp.float32)
        + b1_ref[...]
    )


def kernel(x, edge_index, edge_attr, u, W0, b0, W1, b1):
    ei = edge_index.astype(jnp.int32)
    pad = ((0, PAD_CHUNKS - N_CHUNKS), (0, 0))
    ridx = jnp.pad(ei[0].reshape(N_CHUNKS, CHUNK), pad)
    cidx = jnp.pad(ei[1].reshape(N_CHUNKS, CHUNK), pad)
    zeros = jnp.zeros((STRIPE, D_EDGE), jnp.float32)

    parts = _sc_segment_sums(edge_attr, ridx, cidx, zeros)

    w0c = W0[:, 0:D_EDGE].T
    w0r = W0[:, D_EDGE:2 * D_EDGE].T
    w0x = W0[:, 2 * D_EDGE:2 * D_EDGE + D_FEAT].T
    w0u = W0[:, 2 * D_EDGE + D_FEAT:]
    beff = (b0 + u[0] @ w0u.T).reshape(1, 128)
    w1t = W1.T
    b1r = b1.reshape(1, 128)

    BN = 2000
    grid = (N_NODES // BN,)
    out = pl.pallas_call(
        _mlp_body,
        grid=grid,
        in_specs=[
            pl.BlockSpec((NC, 2, BN, D_EDGE), lambda i: (0, 0, i, 0)),
            pl.BlockSpec((BN, D_FEAT), lambda i: (i, 0)),
            pl.BlockSpec((D_EDGE, 128), lambda i: (0, 0)),
            pl.BlockSpec((D_EDGE, 128), lambda i: (0, 0)),
            pl.BlockSpec((D_FEAT, 128), lambda i: (0, 0)),
            pl.BlockSpec((1, 128), lambda i: (0, 0)),
            pl.BlockSpec((128, 128), lambda i: (0, 0)),
            pl.BlockSpec((1, 128), lambda i: (0, 0)),
        ],
        out_specs=pl.BlockSpec((BN, 128), lambda i: (i, 0)),
        out_shape=jax.ShapeDtypeStruct((N_NODES, 128), jnp.float32),
    )(parts, x, w0c, w0r, w0x, beff, w1t, b1r)
    return out
